# baseline (speedup 1.0000x reference)
import functools

import jax
import jax.numpy as jnp
import numpy as np
from jax import lax
from jax.experimental import pallas as pl
from jax.experimental.pallas import tpu as pltpu
from jax.experimental.pallas import tpu_sc as plsc

N_NODES = 10000
N_EDGES = 160000
D = 256
DW = D // 2

_HI_MASK = np.uint32(0xFFFF0000)

_PM = 2000


def _pack_body(x_ref, out_ref):
    lo = x_ref[:, :DW].astype(jnp.bfloat16).astype(jnp.float32)
    hi = x_ref[:, DW:].astype(jnp.bfloat16).astype(jnp.float32)
    lo_b = lax.bitcast_convert_type(lo, jnp.uint32)
    hi_b = lax.bitcast_convert_type(hi, jnp.uint32)
    w = (lo_b >> 16) | (hi_b & _HI_MASK)
    out_ref[...] = lax.bitcast_convert_type(w, jnp.int32)


def _pack_table(table):
    return pl.pallas_call(
        _pack_body,
        grid=(N_NODES // _PM,),
        in_specs=[pl.BlockSpec((_PM, D), lambda i: (i, 0))],
        out_specs=pl.BlockSpec((_PM, DW), lambda i: (i, 0)),
        out_shape=jax.ShapeDtypeStruct((N_NODES, DW), jnp.int32),
    )(table)


_INFO = plsc.get_sparse_core_info()
_NW = _INFO.num_cores * _INFO.num_subcores
_C = 128
_NGRP = 40
_EPW = _C * _NGRP
E_PAD = _EPW * _NW


def _sc_gather(table_hbm, idx_src_hbm, idx_tgt_hbm, src_out, tgt_out,
               idx_s, idx_t, bs0, bt0, bs1, bt1,
               gs0, gt0, gs1, gt1, ws0, ws1):
    wid = lax.axis_index("s") * _INFO.num_cores + lax.axis_index("c")
    base = wid * _EPW
    pltpu.sync_copy(idx_src_hbm.at[pl.ds(base, _EPW)], idx_s)
    pltpu.sync_copy(idx_tgt_hbm.at[pl.ds(base, _EPW)], idx_t)

    def issue_group(g, bs, bt, gs, gt):
        goff = g * _C
        pltpu.async_copy(table_hbm.at[idx_s.at[pl.ds(goff, _C)]], bs, gs)
        pltpu.async_copy(table_hbm.at[idx_t.at[pl.ds(goff, _C)]], bt, gt)

    def drain_group(bs, bt, gs, gt):
        pltpu.make_async_copy(table_hbm.at[idx_s.at[pl.ds(0, _C)]],
                              bs, gs).wait()
        pltpu.make_async_copy(table_hbm.at[idx_t.at[pl.ds(0, _C)]],
                              bt, gt).wait()

    def write_group(g, bs, bt, wsem):
        row = base + g * _C
        pltpu.async_copy(bs, src_out.at[pl.ds(row, _C)], wsem)
        pltpu.async_copy(bt, tgt_out.at[pl.ds(row, _C)], wsem)
        pltpu.make_async_copy(bs, src_out.at[pl.ds(row, _C)], wsem).wait()
        pltpu.make_async_copy(bt, tgt_out.at[pl.ds(row, _C)], wsem).wait()

    issue_group(0, bs0, bt0, gs0, gt0)

    def body(i, carry):
        g0 = 2 * i
        issue_group(g0 + 1, bs1, bt1, gs1, gt1)
        drain_group(bs0, bt0, gs0, gt0)
        write_group(g0, bs0, bt0, ws0)
        issue_group(g0 + 2, bs0, bt0, gs0, gt0)
        drain_group(bs1, bt1, gs1, gt1)
        write_group(g0 + 1, bs1, bt1, ws1)
        return carry

    lax.fori_loop(0, (_NGRP - 2) // 2, body, 0)

    issue_group(_NGRP - 1, bs1, bt1, gs1, gt1)
    drain_group(bs0, bt0, gs0, gt0)
    write_group(_NGRP - 2, bs0, bt0, ws0)
    drain_group(bs1, bt1, gs1, gt1)
    write_group(_NGRP - 1, bs1, bt1, ws1)


def _gather_rows(table, idx_src, idx_tgt):
    mesh = plsc.VectorSubcoreMesh(core_axis_name="c", subcore_axis_name="s")
    f = functools.partial(
        pl.kernel,
        out_type=[jax.ShapeDtypeStruct((E_PAD, DW), jnp.int32),
                  jax.ShapeDtypeStruct((E_PAD, DW), jnp.int32)],
        mesh=mesh,
        scratch_types=[
            pltpu.VMEM((_EPW,), jnp.int32),
            pltpu.VMEM((_EPW,), jnp.int32),
            pltpu.VMEM((_C, DW), jnp.int32),
            pltpu.VMEM((_C, DW), jnp.int32),
            pltpu.VMEM((_C, DW), jnp.int32),
            pltpu.VMEM((_C, DW), jnp.int32),
            pltpu.SemaphoreType.DMA,
            pltpu.SemaphoreType.DMA,
            pltpu.SemaphoreType.DMA,
            pltpu.SemaphoreType.DMA,
            pltpu.SemaphoreType.DMA,
            pltpu.SemaphoreType.DMA,
        ],
    )(_sc_gather)
    return f(table, idx_src, idx_tgt)


_M = 2000


def _unpack(words):
    w = lax.bitcast_convert_type(words, jnp.uint32)
    lo = lax.bitcast_convert_type(w << 16, jnp.float32).astype(jnp.bfloat16)
    hi = lax.bitcast_convert_type(w & _HI_MASK,
                                  jnp.float32).astype(jnp.bfloat16)
    return jnp.concatenate([lo, hi], axis=-1)


def _mm_body(bond_ref, src_ref, tgt_ref, w1a_ref, w1b_ref, w1c_ref,
             b1_ref, w2_ref, b2_ref, out_ref):
    bond_bf = bond_ref[...].astype(jnp.bfloat16)
    acc = jnp.dot(bond_bf, w1a_ref[...], preferred_element_type=jnp.float32)
    acc = acc + jnp.dot(_unpack(src_ref[...]), w1b_ref[...],
                        preferred_element_type=jnp.float32)
    acc = acc + jnp.dot(_unpack(tgt_ref[...]), w1c_ref[...],
                        preferred_element_type=jnp.float32)
    h = jnp.maximum(acc + b1_ref[...], 0.0).astype(jnp.bfloat16)
    out_ref[...] = jnp.dot(h, w2_ref[...],
                           preferred_element_type=jnp.float32) + b2_ref[...]


def _dense(bond, src_w, tgt_w, w1a, w1b, w1c, b1, w2, b2):
    grid = (N_EDGES // _M,)
    row_spec = pl.BlockSpec((_M, D), lambda i: (i, 0))
    word_spec = pl.BlockSpec((_M, DW), lambda i: (i, 0))
    full = lambda shape: pl.BlockSpec(shape, lambda i: (0, 0))
    return pl.pallas_call(
        _mm_body,
        grid=grid,
        in_specs=[row_spec, word_spec, word_spec,
                  full((D, 2 * D)), full((D, 2 * D)), full((D, 2 * D)),
                  full((1, 2 * D)), full((2 * D, D)), full((1, D))],
        out_specs=row_spec,
        out_shape=jax.ShapeDtypeStruct((N_EDGES, D), jnp.float32),
    )(bond, src_w, tgt_w, w1a, w1b, w1c, b1, w2, b2)


def kernel(atom_state, bond_state, connectivity, W1, b1, W2, b2):
    table_words = _pack_table(atom_state[0])
    bond = bond_state[0]
    idx_tgt = jnp.pad(connectivity[0, :, 0], (0, E_PAD - N_EDGES))
    idx_src = jnp.pad(connectivity[0, :, 1], (0, E_PAD - N_EDGES))
    src_w, tgt_w = _gather_rows(table_words, idx_src, idx_tgt)
    W1_bf = W1.astype(jnp.bfloat16)
    w1a, w1b, w1c = W1_bf[:D], W1_bf[D:2 * D], W1_bf[2 * D:]
    out = _dense(bond, src_w, tgt_w, w1a, w1b, w1c,
                 b1.reshape(1, 2 * D), W2.astype(jnp.bfloat16),
                 b2.reshape(1, D))
    return out[None]


# speedup vs baseline: 2.3019x; 2.3019x over previous
import functools

import jax
import jax.numpy as jnp
import numpy as np
from jax import lax
from jax.experimental import pallas as pl
from jax.experimental.pallas import tpu as pltpu
from jax.experimental.pallas import tpu_sc as plsc

N_NODES = 10000
N_EDGES = 160000
D = 256
DW = D // 2

_HI_MASK = np.uint32(0xFFFF0000)

_S = 5
_SLAB = N_EDGES // _S

_PM = 2000


def _pack_body(x_ref, out_ref):
    lo = x_ref[:, :DW].astype(jnp.bfloat16).astype(jnp.float32)
    hi = x_ref[:, DW:].astype(jnp.bfloat16).astype(jnp.float32)
    lo_b = lax.bitcast_convert_type(lo, jnp.uint32)
    hi_b = lax.bitcast_convert_type(hi, jnp.uint32)
    w = (lo_b >> 16) | (hi_b & _HI_MASK)
    out_ref[...] = lax.bitcast_convert_type(w, jnp.int32)


def _pack_table(table):
    return pl.pallas_call(
        _pack_body,
        grid=(N_NODES // _PM,),
        in_specs=[pl.BlockSpec((_PM, D), lambda i: (i, 0))],
        out_specs=pl.BlockSpec((_PM, DW), lambda i: (i, 0)),
        out_shape=jax.ShapeDtypeStruct((N_NODES, DW), jnp.int32),
    )(table)


_INFO = plsc.get_sparse_core_info()
_NW = _INFO.num_cores * _INFO.num_subcores
_EPW = _SLAB // _NW
_C = 40
_G = 5
_GR = _C * _G
_NGRP = _EPW // _GR


def _sc_gather(table_hbm, idx_src_hbm, idx_tgt_hbm, src_out, tgt_out,
               idx_s, idx_t, bs0, bt0, bs1, bt1,
               gs0, gt0, gs1, gt1, ws0, ws1):
    wid = lax.axis_index("s") * _INFO.num_cores + lax.axis_index("c")
    base = wid * _EPW
    pltpu.sync_copy(idx_src_hbm.at[pl.ds(base, _EPW)], idx_s)
    pltpu.sync_copy(idx_tgt_hbm.at[pl.ds(base, _EPW)], idx_t)

    def issue_g(goff, idx_v, buf, sem):
        for j in range(_G):
            pltpu.async_copy(
                table_hbm.at[idx_v.at[pl.ds(goff + j * _C, _C)]],
                buf.at[pl.ds(j * _C, _C)], sem)

    def wait_g(idx_v, buf, sem):
        for j in range(_G):
            pltpu.make_async_copy(
                table_hbm.at[idx_v.at[pl.ds(j * _C, _C)]],
                buf.at[pl.ds(j * _C, _C)], sem).wait()

    def issue_group(g, bs, bt, gs, gt):
        goff = g * _GR
        issue_g(goff, idx_s, bs, gs)
        issue_g(goff, idx_t, bt, gt)

    def drain_group(bs, bt, gs, gt):
        wait_g(idx_s, bs, gs)
        wait_g(idx_t, bt, gt)

    def write_group(g, bs, bt, wsem):
        row = base + g * _GR
        pltpu.async_copy(bs, src_out.at[pl.ds(row, _GR)], wsem)
        pltpu.async_copy(bt, tgt_out.at[pl.ds(row, _GR)], wsem)

    def wait_writes(g, bs, bt, wsem):
        row = base + g * _GR
        pltpu.make_async_copy(bs, src_out.at[pl.ds(row, _GR)], wsem).wait()
        pltpu.make_async_copy(bt, tgt_out.at[pl.ds(row, _GR)], wsem).wait()

    issue_group(0, bs0, bt0, gs0, gt0)

    def body(i, carry):
        g0 = 2 * i
        issue_group(g0 + 1, bs1, bt1, gs1, gt1)
        drain_group(bs0, bt0, gs0, gt0)
        write_group(g0, bs0, bt0, ws0)
        wait_writes(g0, bs0, bt0, ws0)
        issue_group(g0 + 2, bs0, bt0, gs0, gt0)
        drain_group(bs1, bt1, gs1, gt1)
        write_group(g0 + 1, bs1, bt1, ws1)
        wait_writes(g0 + 1, bs1, bt1, ws1)
        return carry

    lax.fori_loop(0, (_NGRP - 1) // 2, body, 0)

    drain_group(bs0, bt0, gs0, gt0)
    write_group(_NGRP - 1, bs0, bt0, ws0)
    wait_writes(_NGRP - 1, bs0, bt0, ws0)


def _gather_rows(table, idx_src, idx_tgt):
    mesh = plsc.VectorSubcoreMesh(core_axis_name="c", subcore_axis_name="s")
    f = functools.partial(
        pl.kernel,
        out_type=[jax.ShapeDtypeStruct((_SLAB, DW), jnp.int32),
                  jax.ShapeDtypeStruct((_SLAB, DW), jnp.int32)],
        mesh=mesh,
        scratch_types=[
            pltpu.VMEM((_EPW,), jnp.int32),
            pltpu.VMEM((_EPW,), jnp.int32),
            pltpu.VMEM((_GR, DW), jnp.int32),
            pltpu.VMEM((_GR, DW), jnp.int32),
            pltpu.VMEM((_GR, DW), jnp.int32),
            pltpu.VMEM((_GR, DW), jnp.int32),
            pltpu.SemaphoreType.DMA,
            pltpu.SemaphoreType.DMA,
            pltpu.SemaphoreType.DMA,
            pltpu.SemaphoreType.DMA,
            pltpu.SemaphoreType.DMA,
            pltpu.SemaphoreType.DMA,
        ],
    )(_sc_gather)
    return f(table, idx_src, idx_tgt)


_M = 4000
_SPS = _SLAB // _M


def _unpack(words):
    w = lax.bitcast_convert_type(words, jnp.uint32)
    lo = lax.bitcast_convert_type(w << 16, jnp.float32).astype(jnp.bfloat16)
    hi = lax.bitcast_convert_type(w & _HI_MASK,
                                  jnp.float32).astype(jnp.bfloat16)
    return jnp.concatenate([lo, hi], axis=-1)


def _mm_body(bond_ref, src_ref, tgt_ref, w1a_ref, w1b_ref, w1c_ref,
             b1_ref, w2_ref, b2_ref, out_ref):
    bond_bf = bond_ref[...].astype(jnp.bfloat16)
    acc = jnp.dot(bond_bf, w1a_ref[...], preferred_element_type=jnp.float32)
    acc = acc + jnp.dot(_unpack(src_ref[...]), w1b_ref[...],
                        preferred_element_type=jnp.float32)
    acc = acc + jnp.dot(_unpack(tgt_ref[...]), w1c_ref[...],
                        preferred_element_type=jnp.float32)
    h = jnp.maximum(acc + b1_ref[...], 0.0).astype(jnp.bfloat16)
    out_ref[...] = jnp.dot(h, w2_ref[...],
                           preferred_element_type=jnp.float32) + b2_ref[...]


def _mm_body_acc(acc_ref, *rest):
    del acc_ref
    _mm_body(*rest)


def _dense_slab(s, prev_out, bond, src_w, tgt_w,
                w1a, w1b, w1c, b1, w2, b2):
    row_spec = pl.BlockSpec((_M, D), lambda i, s=s: (s * _SPS + i, 0))
    word_spec = pl.BlockSpec((_M, DW), lambda i: (i, 0))
    full = lambda shape: pl.BlockSpec(shape, lambda i: (0, 0))
    main_specs = [row_spec, word_spec, word_spec,
                  full((D, 2 * D)), full((D, 2 * D)), full((D, 2 * D)),
                  full((1, 2 * D)), full((2 * D, D)), full((1, D))]
    args = (bond, src_w, tgt_w, w1a, w1b, w1c, b1, w2, b2)
    if prev_out is None:
        return pl.pallas_call(
            _mm_body,
            grid=(_SPS,),
            in_specs=main_specs,
            out_specs=row_spec,
            out_shape=jax.ShapeDtypeStruct((N_EDGES, D), jnp.float32),
        )(*args)
    return pl.pallas_call(
        _mm_body_acc,
        grid=(_SPS,),
        in_specs=[pl.BlockSpec(memory_space=pl.ANY)] + main_specs,
        out_specs=row_spec,
        out_shape=jax.ShapeDtypeStruct((N_EDGES, D), jnp.float32),
        input_output_aliases={0: 0},
    )(prev_out, *args)


def kernel(atom_state, bond_state, connectivity, W1, b1, W2, b2):
    table_words = _pack_table(atom_state[0])
    bond = bond_state[0]
    idx_tgt = connectivity[0, :, 0]
    idx_src = connectivity[0, :, 1]
    W1_bf = W1.astype(jnp.bfloat16)
    w1a, w1b, w1c = W1_bf[:D], W1_bf[D:2 * D], W1_bf[2 * D:]
    b1r = b1.reshape(1, 2 * D)
    W2_bf = W2.astype(jnp.bfloat16)
    b2r = b2.reshape(1, D)

    gathered = []
    for s in range(_S):
        lo = s * _SLAB
        gathered.append(_gather_rows(
            table_words,
            lax.slice_in_dim(idx_src, lo, lo + _SLAB),
            lax.slice_in_dim(idx_tgt, lo, lo + _SLAB)))

    out = None
    for s in range(_S):
        src_w, tgt_w = gathered[s]
        out = _dense_slab(s, out, bond, src_w, tgt_w,
                          w1a, w1b, w1c, b1r, W2_bf, b2r)
    return out[None]
